# baseline (speedup 1.0000x reference)
import functools

import jax
import jax.numpy as jnp
from jax import lax
from jax.experimental import pallas as pl
from jax.experimental.pallas import tpu as pltpu

F = 26
K = 16


def _body(x_ref, ev_ref, w_ref, b_ref, o_ref, a_scr):
    @pl.when(pl.program_id(0) == 0)
    def _():
        ev = ev_ref[...]
        s = jnp.zeros((F, F), jnp.float32)
        for k in range(K):
            sk = ev[:, k, :]
            s = s + sk * sk.T
        ri = lax.broadcasted_iota(jnp.int32, (F, F), 0)
        ci = lax.broadcasted_iota(jnp.int32, (F, F), 1)
        a_scr[...] = jnp.where(ri == ci, 0.0, s) * 0.5

    x = x_ref[...]
    t = jnp.dot(x, a_scr[...], preferred_element_type=jnp.float32)
    q = (x * t) + x * w_ref[...]
    ones = jnp.ones((F, 1), jnp.float32)
    o_ref[...] = jnp.dot(q, ones, preferred_element_type=jnp.float32) + b_ref[0, 0]


@functools.lru_cache(maxsize=None)
def _build(B, BT):
    return pl.pallas_call(
        _body,
        grid=(B // BT,),
        in_specs=[
            pl.BlockSpec((BT, F), lambda j: (j, 0)),
            pl.BlockSpec((F, K, F), lambda j: (0, 0, 0)),
            pl.BlockSpec((1, F), lambda j: (0, 0)),
            pl.BlockSpec((1, 1), lambda j: (0, 0)),
        ],
        out_specs=pl.BlockSpec((BT, 1), lambda j: (j, 0)),
        out_shape=jax.ShapeDtypeStruct((B, 1), jnp.float32),
        scratch_shapes=[pltpu.VMEM((F, F), jnp.float32)],
    )


def kernel(x, emb, W, b):
    B = x.shape[0]
    bt = min(B, 4096)
    return _build(B, bt)(x, emb.transpose(0, 2, 1), W, b.reshape(1, 1))


# speedup vs baseline: 1.1880x; 1.1880x over previous
import functools

import jax
import jax.numpy as jnp
from jax import lax
from jax.experimental import pallas as pl
from jax.experimental.pallas import tpu as pltpu

F = 26
K = 16


def _body(x_ref, ev_ref, w_ref, b_ref, o_ref, a_scr):
    @pl.when(pl.program_id(0) == 0)
    def _():
        ev = ev_ref[...]
        s = jnp.zeros((F, F), jnp.float32)
        for k in range(K):
            sk = ev[:, k, :]
            s = s + sk * sk.T
        ri = lax.broadcasted_iota(jnp.int32, (F, F), 0)
        ci = lax.broadcasted_iota(jnp.int32, (F, F), 1)
        a_scr[...] = jnp.where(ri == ci, 0.0, s) * 0.5

    x = x_ref[...]
    t = jnp.dot(x, a_scr[...], preferred_element_type=jnp.float32)
    q = (x * t) + x * w_ref[...]
    ones = jnp.ones((F, 1), jnp.float32)
    r = jnp.dot(q, ones, preferred_element_type=jnp.float32) + b_ref[0, 0]
    o_ref[...] = r[:, 0]


@functools.lru_cache(maxsize=None)
def _build(B, BT):
    return pl.pallas_call(
        _body,
        grid=(B // BT,),
        in_specs=[
            pl.BlockSpec((BT, F), lambda j: (j, 0)),
            pl.BlockSpec((F, K, F), lambda j: (0, 0, 0)),
            pl.BlockSpec((1, F), lambda j: (0, 0)),
            pl.BlockSpec((1, 1), lambda j: (0, 0)),
        ],
        out_specs=pl.BlockSpec((BT,), lambda j: (j,)),
        out_shape=jax.ShapeDtypeStruct((B,), jnp.float32),
        scratch_shapes=[pltpu.VMEM((F, F), jnp.float32)],
    )


def kernel(x, emb, W, b):
    B = x.shape[0]
    bt = min(B, 4096)
    out = _build(B, bt)(x, emb.transpose(0, 2, 1), W, b.reshape(1, 1))
    return out[:, None]


# speedup vs baseline: 3.1344x; 2.6383x over previous
import functools

import jax
import jax.numpy as jnp
from jax import lax
from jax.experimental import pallas as pl
from jax.experimental.pallas import tpu as pltpu

F = 26
K = 16


def _body(xt_ref, ev_ref, w_ref, b_ref, o_ref, a_scr):
    @pl.when(pl.program_id(0) == 0)
    def _():
        ev = ev_ref[...]
        s = jnp.zeros((F, F), jnp.float32)
        for k in range(K):
            sk = ev[:, k, :]
            s = s + sk * sk.T
        ri = lax.broadcasted_iota(jnp.int32, (F, F), 0)
        ci = lax.broadcasted_iota(jnp.int32, (F, F), 1)
        a_scr[...] = jnp.where(ri == ci, 0.0, s) * 0.5

    xt = xt_ref[...]
    ax = jnp.dot(a_scr[...], xt, preferred_element_type=jnp.float32)
    xw = jnp.dot(w_ref[...], xt, preferred_element_type=jnp.float32)
    o_ref[...] = jnp.sum(xt * ax, axis=0) + xw[0] + b_ref[0, 0]


@functools.lru_cache(maxsize=None)
def _build(B, BT):
    return pl.pallas_call(
        _body,
        grid=(B // BT,),
        in_specs=[
            pl.BlockSpec((F, BT), lambda j: (0, j)),
            pl.BlockSpec((F, K, F), lambda j: (0, 0, 0)),
            pl.BlockSpec((1, F), lambda j: (0, 0)),
            pl.BlockSpec((1, 1), lambda j: (0, 0)),
        ],
        out_specs=pl.BlockSpec((BT,), lambda j: (j,)),
        out_shape=jax.ShapeDtypeStruct((B,), jnp.float32),
        scratch_shapes=[pltpu.VMEM((F, F), jnp.float32)],
    )


def kernel(x, emb, W, b):
    B = x.shape[0]
    bt = min(B, 2048)
    out = _build(B, bt)(x.T, emb.transpose(0, 2, 1), W, b.reshape(1, 1))
    return out[:, None]


# speedup vs baseline: 5.7868x; 1.8462x over previous
import functools

import jax
import jax.numpy as jnp
from jax import lax
from jax.experimental import pallas as pl
from jax.experimental.pallas import tpu as pltpu

F = 26
K = 16


def _body(xt_ref, ev_ref, w_ref, b_ref, o_ref, a_scr):
    @pl.when(pl.program_id(0) == 0)
    def _():
        ev = ev_ref[...]
        s = jnp.zeros((F, F), jnp.float32)
        for k in range(K):
            sk = ev[:, k, :]
            s = s + sk * sk.T
        ri = lax.broadcasted_iota(jnp.int32, (F, F), 0)
        ci = lax.broadcasted_iota(jnp.int32, (F, F), 1)
        a_scr[...] = jnp.where(ri == ci, 0.0, s) * 0.5

    xt = xt_ref[...]
    ax = jnp.dot(a_scr[...], xt, preferred_element_type=jnp.float32)
    xw = jnp.dot(w_ref[...], xt, preferred_element_type=jnp.float32)
    o_ref[...] = jnp.sum(xt * ax, axis=0) + xw[0] + b_ref[0, 0]


@functools.lru_cache(maxsize=None)
def _build(B, BT):
    return pl.pallas_call(
        _body,
        grid=(B // BT,),
        in_specs=[
            pl.BlockSpec((F, BT), lambda j: (0, j)),
            pl.BlockSpec((F, K, F), lambda j: (0, 0, 0)),
            pl.BlockSpec((1, F), lambda j: (0, 0)),
            pl.BlockSpec((1, 1), lambda j: (0, 0)),
        ],
        out_specs=pl.BlockSpec((BT,), lambda j: (j,)),
        out_shape=jax.ShapeDtypeStruct((B,), jnp.float32),
        scratch_shapes=[pltpu.VMEM((F, F), jnp.float32)],
    )


def kernel(x, emb, W, b):
    B = x.shape[0]
    bt = min(B, 8192)
    out = _build(B, bt)(x.T, emb.transpose(0, 2, 1), W, b.reshape(1, 1))
    return out[:, None]
